# baseline (speedup 1.0000x reference)
import jax
import jax.numpy as jnp
from jax.experimental import pallas as pl
from jax.experimental.pallas import tpu as pltpu

_N = 10000
_DIN = 256
_DOUT = 256

_BM = 200


def _gcn_body(adj_a_ref, adj_b_ref, xf_ref, w_ref, out_ref, s_ref):
    @pl.when(pl.program_id(0) == 0)
    def _():
        s_ref[...] = jnp.dot(xf_ref[...], w_ref[...],
                             preferred_element_type=jnp.float32)

    out_ref[:_BM, :] = jnp.dot(adj_a_ref[...], s_ref[...],
                               preferred_element_type=jnp.float32)
    out_ref[_BM:, :] = jnp.dot(adj_b_ref[...], s_ref[...],
                               preferred_element_type=jnp.float32)


@jax.jit
def kernel(x, adj, W):
    xf = x.reshape(_N, _DIN)
    out = pl.pallas_call(
        _gcn_body,
        grid=(_N // (2 * _BM),),
        in_specs=[
            pl.BlockSpec((_BM, _N), lambda i: (2 * i, 0)),
            pl.BlockSpec((_BM, _N), lambda i: (2 * i + 1, 0)),
            pl.BlockSpec((_N, _DIN), lambda i: (0, 0)),
            pl.BlockSpec((_DIN, _DOUT), lambda i: (0, 0)),
        ],
        out_specs=pl.BlockSpec((2 * _BM, _DOUT), lambda i: (i, 0)),
        out_shape=jax.ShapeDtypeStruct((_N, _DOUT), jnp.float32),
        scratch_shapes=[pltpu.VMEM((_N, _DOUT), jnp.float32)],
    )(adj, adj, xf, W)
    return out


# speedup vs baseline: 1.0098x; 1.0098x over previous
import jax
import jax.numpy as jnp
from jax.experimental import pallas as pl
from jax.experimental.pallas import tpu as pltpu

_N = 10000
_DIN = 256
_DOUT = 256

_BM = 400


def _gcn_body(adj_ref, xf_ref, w_ref, out_ref, s_ref):
    @pl.when(pl.program_id(0) == 0)
    def _():
        s_ref[...] = jnp.dot(xf_ref[...], w_ref[...],
                             preferred_element_type=jnp.float32
                             ).astype(jnp.bfloat16)

    out_ref[...] = jnp.dot(adj_ref[...].astype(jnp.bfloat16), s_ref[...],
                           preferred_element_type=jnp.float32)


@jax.jit
def kernel(x, adj, W):
    xf = x.reshape(_N, _DIN)
    out = pl.pallas_call(
        _gcn_body,
        grid=(_N // _BM,),
        in_specs=[
            pl.BlockSpec((_BM, _N), lambda i: (i, 0)),
            pl.BlockSpec((_N, _DIN), lambda i: (0, 0)),
            pl.BlockSpec((_DIN, _DOUT), lambda i: (0, 0)),
        ],
        out_specs=pl.BlockSpec((_BM, _DOUT), lambda i: (i, 0)),
        out_shape=jax.ShapeDtypeStruct((_N, _DOUT), jnp.float32),
        scratch_shapes=[pltpu.VMEM((_N, _DOUT), jnp.bfloat16)],
    )(adj, xf, W)
    return out
